# baseline (speedup 1.0000x reference)
import math

import jax
import jax.numpy as jnp
from jax.experimental import pallas as pl
from jax.experimental.pallas import tpu as pltpu

D = 768
F = 3072
E = 8
K = 2
H = 4
DH = D // H
N = 2048
TT = 256
NT = N // TT
CO = 4
FC = F // CO
EPS = 1e-5
HIGH = jax.lax.Precision.HIGHEST


def _dott(a, w, precision=None):
    return jax.lax.dot_general(
        a, w, (((1,), (1,)), ((), ())),
        precision=precision, preferred_element_type=jnp.float32)


def _lnorm(v, g, b):
    m = jnp.mean(v, axis=-1, keepdims=True)
    var = jnp.mean((v - m) ** 2, axis=-1, keepdims=True)
    return (v - m) / jnp.sqrt(var + EPS) * g + b


def _gelu(v):
    return 0.5 * v * (1.0 + jax.lax.erf(v * (1.0 / math.sqrt(2.0))))


def _attn_body(xf_ref, xt_ref, wq_ref, wk_ref, wv_ref, bq_ref, bk_ref,
               bv_ref, osum_ref, ks_ref, vs_ref):
    qt = pl.program_id(1)

    @pl.when(qt == 0)
    def _():
        xf = xf_ref[...]
        ks_ref[...] = _dott(xf, wk_ref[0], HIGH) + bk_ref[0]
        vs_ref[...] = _dott(xf, wv_ref[0], HIGH) + bv_ref[0]

    q = _dott(xt_ref[...], wq_ref[0], HIGH) + bq_ref[0]
    s = jax.lax.dot_general(q, ks_ref[...], (((1,), (1,)), ((), ())),
                            precision=HIGH,
                            preferred_element_type=jnp.float32)
    s = s * (1.0 / math.sqrt(DH))
    s = s - jnp.max(s, axis=-1, keepdims=True)
    ex = jnp.exp(s)
    prob = ex / jnp.sum(ex, axis=-1, keepdims=True)
    o = jax.lax.dot_general(prob, vs_ref[...], (((1,), (0,)), ((), ())),
                            precision=HIGH,
                            preferred_element_type=jnp.float32)
    part = jnp.sum(o, axis=0, keepdims=True)

    @pl.when(qt == 0)
    def _():
        osum_ref[0] = part

    @pl.when(qt != 0)
    def _():
        osum_ref[0] = osum_ref[0] + part


def _attn(xf, wqh, wkh, wvh, bqh, bkh, bvh):
    return pl.pallas_call(
        _attn_body,
        grid=(H, NT),
        in_specs=[
            pl.BlockSpec((N, D), lambda h, t: (0, 0)),
            pl.BlockSpec((TT, D), lambda h, t: (t, 0)),
            pl.BlockSpec((1, DH, D), lambda h, t: (h, 0, 0)),
            pl.BlockSpec((1, DH, D), lambda h, t: (h, 0, 0)),
            pl.BlockSpec((1, DH, D), lambda h, t: (h, 0, 0)),
            pl.BlockSpec((1, 1, DH), lambda h, t: (h, 0, 0)),
            pl.BlockSpec((1, 1, DH), lambda h, t: (h, 0, 0)),
            pl.BlockSpec((1, 1, DH), lambda h, t: (h, 0, 0)),
        ],
        out_specs=pl.BlockSpec((1, 1, DH), lambda h, t: (h, 0, 0)),
        out_shape=jax.ShapeDtypeStruct((H, 1, DH), jnp.float32),
        scratch_shapes=[pltpu.VMEM((N, DH), jnp.float32),
                        pltpu.VMEM((N, DH), jnp.float32)],
    )(xf, xf, wqh, wkh, wvh, bqh, bkh, bvh)


def _gate_body(x_ref, os_ref, wo_ref, bo_ref, g1w_ref, g1b_ref, glg_ref,
               glb_ref, g2w_ref, g2b_ref, g3w_ref, g3b_ref, t1w_ref,
               t1b_ref, t2w_ref, t2b_ref, rw_ref,
               topi_ref, a_ref, c_ref, loss_ref):
    xmean = jnp.mean(x_ref[...], axis=0, keepdims=True)
    pooled = _dott(os_ref[...] * (1.0 / N), wo_ref[...], HIGH) \
        + bo_ref[...] + xmean
    h = _gelu(_dott(pooled, g1w_ref[...], HIGH) + g1b_ref[...])
    h = _lnorm(h, glg_ref[...], glb_ref[...])
    h = _gelu(_dott(h, g2w_ref[...], HIGH) + g2b_ref[...])
    logits = _dott(h, g3w_ref[...], HIGH) + g3b_ref[...]

    t = _gelu(_dott(pooled, t1w_ref[...], HIGH) + t1b_ref[...])
    traw = jnp.sum(t * t2w_ref[...]) + t2b_ref[0]
    temp = jnp.clip(jnp.logaddexp(traw, 0.0), 0.1, 5.0)
    logits = logits / temp

    io8 = jax.lax.broadcasted_iota(jnp.int32, (1, E), 1)
    v1 = jnp.max(logits)
    i1 = jnp.min(jnp.where(logits == v1, io8, E))
    masked = jnp.where(io8 == i1, -jnp.inf, logits)
    v2 = jnp.max(masked)
    i2 = jnp.min(jnp.where(masked == v2, io8, E))

    e2 = jnp.exp(v2 - v1)
    den = 1.0 + e2
    g1v = 1.0 / den
    g2v = e2 / den

    ex = jnp.exp(logits - v1)
    probs = ex / jnp.sum(ex)
    mu = jnp.mean(probs)
    var_loss = jnp.sum((probs - mu) ** 2) / (E - 1) * E
    ent = -jnp.sum(probs * jnp.log(probs + 1e-8))
    load = (var_loss + 0.1 * (math.log(E) - ent)) * 0.01

    rw = rw_ref[...]
    rw1 = jnp.sum(jnp.where(io8 == i1, rw, 0.0))
    rw2 = jnp.sum(jnp.where(io8 == i2, rw, 0.0))

    io2 = jax.lax.broadcasted_iota(jnp.int32, (1, K), 1)
    topi_ref[...] = jnp.where(io2 == 0, i1, i2).astype(jnp.int32)
    a_ref[...] = jnp.where(io2 == 0, g1v * rw1, g2v * rw2)
    c_ref[...] = jnp.reshape(g1v * (1.0 - rw1) + g2v * (1.0 - rw2), (1, 1))
    loss_ref[...] = jnp.reshape(load, (1, 1))


def _gate(xf, osum2, p):
    full = lambda shape: pl.BlockSpec(shape, lambda: tuple(0 for _ in shape))
    args = [xf, osum2, p['attn_Wo'], p['attn_bo'].reshape(1, D),
            p['g1_W'], p['g1_b'].reshape(1, D),
            p['g_ln_g'].reshape(1, D), p['g_ln_b'].reshape(1, D),
            p['g2_W'], p['g2_b'].reshape(1, D // 2),
            p['g3_W'], p['g3_b'].reshape(1, E),
            p['t1_W'], p['t1_b'].reshape(1, D // 4),
            p['t2_W'], p['t2_b'].reshape(1,),
            p['rw'].reshape(1, E)]
    in_specs = [full(a.shape) for a in args]
    in_specs[15] = pl.BlockSpec(memory_space=pltpu.SMEM)
    return pl.pallas_call(
        _gate_body,
        grid=(),
        in_specs=in_specs,
        out_specs=[full((1, K)), full((1, K)), full((1, 1)), full((1, 1))],
        out_shape=[jax.ShapeDtypeStruct((1, K), jnp.int32),
                   jax.ShapeDtypeStruct((1, K), jnp.float32),
                   jax.ShapeDtypeStruct((1, 1), jnp.float32),
                   jax.ShapeDtypeStruct((1, 1), jnp.float32)],
    )(*args)


def _wc_body(s_ref, w_ref, o_ref):
    o_ref[0] = w_ref[0].astype(jnp.bfloat16)


def _wcast(topi, w, rows):
    _, R, C = w.shape
    nc = R // rows
    grid_spec = pltpu.PrefetchScalarGridSpec(
        num_scalar_prefetch=1,
        grid=(K, nc),
        in_specs=[pl.BlockSpec((1, rows, C), lambda k, c, s: (s[k], c, 0))],
        out_specs=pl.BlockSpec((1, rows, C), lambda k, c, s: (k, c, 0)),
    )
    return pl.pallas_call(
        _wc_body,
        grid_spec=grid_spec,
        out_shape=jax.ShapeDtypeStruct((K, R, C), jnp.bfloat16),
    )(topi, w)


def _bf(v):
    return v.astype(jnp.bfloat16)


def _ka_body(s_ref, x_ref, itw_ref, itb_ref, itg_ref, itb2_ref,
             l0w_ref, l0b_ref, l0g_ref, l0b2_ref,
             ig1w_ref, ig1b_ref, ig2w_ref, ig2b_ref, h1_ref, ig_ref):
    xt = _bf(x_ref[...])
    h0 = _dott(xt, itw_ref[0]) + itb_ref[0]
    h0 = _gelu(_lnorm(h0, itg_ref[0], itb2_ref[0]))
    h1 = _dott(_bf(h0), l0w_ref[0]) + l0b_ref[0]
    h1_ref[0] = _bf(_lnorm(_gelu(h1), l0g_ref[0], l0b2_ref[0]))
    t = _gelu(_dott(xt, ig1w_ref[0]) + ig1b_ref[0])
    ig_ref[0] = _bf(jax.nn.sigmoid(_dott(_bf(t), ig2w_ref[0]) + ig2b_ref[0]))


def _ka(topi, xf, itw, l0w, ig1w, ig2w, p):
    grid_spec = pltpu.PrefetchScalarGridSpec(
        num_scalar_prefetch=1,
        grid=(K, NT),
        in_specs=[
            pl.BlockSpec((TT, D), lambda k, t, s: (t, 0)),
            pl.BlockSpec((1, D, D), lambda k, t, s: (k, 0, 0)),
            pl.BlockSpec((1, 1, D), lambda k, t, s: (s[k], 0, 0)),
            pl.BlockSpec((1, 1, D), lambda k, t, s: (s[k], 0, 0)),
            pl.BlockSpec((1, 1, D), lambda k, t, s: (s[k], 0, 0)),
            pl.BlockSpec((1, F, D), lambda k, t, s: (k, 0, 0)),
            pl.BlockSpec((1, 1, F), lambda k, t, s: (s[k], 0, 0)),
            pl.BlockSpec((1, 1, F), lambda k, t, s: (s[k], 0, 0)),
            pl.BlockSpec((1, 1, F), lambda k, t, s: (s[k], 0, 0)),
            pl.BlockSpec((1, D // 4, D), lambda k, t, s: (k, 0, 0)),
            pl.BlockSpec((1, 1, D // 4), lambda k, t, s: (s[k], 0, 0)),
            pl.BlockSpec((1, D, D // 4), lambda k, t, s: (k, 0, 0)),
            pl.BlockSpec((1, 1, D), lambda k, t, s: (s[k], 0, 0)),
        ],
        out_specs=[pl.BlockSpec((1, TT, F), lambda k, t, s: (k, t, 0)),
                   pl.BlockSpec((1, TT, D), lambda k, t, s: (k, t, 0))],
    )
    return pl.pallas_call(
        _ka_body,
        grid_spec=grid_spec,
        out_shape=[jax.ShapeDtypeStruct((K, N, F), jnp.bfloat16),
                   jax.ShapeDtypeStruct((K, N, D), jnp.bfloat16)],
    )(topi, xf, itw, p['it_b'].reshape(E, 1, D),
      p['it_ln_g'].reshape(E, 1, D), p['it_ln_b'].reshape(E, 1, D),
      l0w, p['l0_b'].reshape(E, 1, F),
      p['l0_ln_g'].reshape(E, 1, F), p['l0_ln_b'].reshape(E, 1, F),
      ig1w, p['ig1_b'].reshape(E, 1, D // 4),
      ig2w, p['ig2_b'].reshape(E, 1, D))


def _kl1_body(s_ref, h1_ref, w_ref, b_ref, out_ref):
    out_ref[0] = _bf(_dott(h1_ref[0], w_ref[0]) + b_ref[0])


def _kl1(topi, h1, l1w, p):
    grid_spec = pltpu.PrefetchScalarGridSpec(
        num_scalar_prefetch=1,
        grid=(K, CO, NT),
        in_specs=[
            pl.BlockSpec((1, TT, F), lambda k, c, t, s: (k, t, 0)),
            pl.BlockSpec((1, FC, F), lambda k, c, t, s: (k, c, 0)),
            pl.BlockSpec((1, 1, FC), lambda k, c, t, s: (s[k], 0, c)),
        ],
        out_specs=pl.BlockSpec((1, TT, FC), lambda k, c, t, s: (k, t, c)),
    )
    return pl.pallas_call(
        _kl1_body,
        grid_spec=grid_spec,
        out_shape=jax.ShapeDtypeStruct((K, N, F), jnp.bfloat16),
    )(topi, h1, l1w, p['l1_b'].reshape(E, 1, F))


def _kop_body(s_ref, h2p_ref, l1g_ref, l1b2_ref, opw_ref, opb_ref,
              ig_ref, es_ref, eb_ref, a_ref, z_ref):
    h2 = _lnorm(_gelu(h2p_ref[0].astype(jnp.float32)), l1g_ref[0], l1b2_ref[0])
    o = _dott(_bf(h2), opw_ref[0]) + opb_ref[0]
    o = o * ig_ref[0].astype(jnp.float32) * es_ref[0] + eb_ref[0]
    k = pl.program_id(0)
    z_ref[0] = a_ref[0, k] * o


def _kop(topi, h2pre, igx, avec, opw, p):
    grid_spec = pltpu.PrefetchScalarGridSpec(
        num_scalar_prefetch=1,
        grid=(K, NT),
        in_specs=[
            pl.BlockSpec((1, TT, F), lambda k, t, s: (k, t, 0)),
            pl.BlockSpec((1, 1, F), lambda k, t, s: (s[k], 0, 0)),
            pl.BlockSpec((1, 1, F), lambda k, t, s: (s[k], 0, 0)),
            pl.BlockSpec((1, D, F), lambda k, t, s: (k, 0, 0)),
            pl.BlockSpec((1, 1, D), lambda k, t, s: (s[k], 0, 0)),
            pl.BlockSpec((1, TT, D), lambda k, t, s: (k, t, 0)),
            pl.BlockSpec((1, 1, D), lambda k, t, s: (s[k], 0, 0)),
            pl.BlockSpec((1, 1, D), lambda k, t, s: (s[k], 0, 0)),
            pl.BlockSpec(memory_space=pltpu.SMEM),
        ],
        out_specs=pl.BlockSpec((1, TT, D), lambda k, t, s: (k, t, 0)),
    )
    return pl.pallas_call(
        _kop_body,
        grid_spec=grid_spec,
        out_shape=jax.ShapeDtypeStruct((K, N, D), jnp.float32),
    )(topi, h2pre, p['l1_ln_g'].reshape(E, 1, F),
      p['l1_ln_b'].reshape(E, 1, F),
      opw, p['op_b'].reshape(E, 1, D),
      igx, p['es'].reshape(E, 1, D), p['eb'].reshape(E, 1, D), avec)


def _kf_body(z_ref, x_ref, c_ref, fw_ref, fb_ref, ng_ref, nb_ref, y_ref):
    xt = x_ref[...]
    comb = z_ref[0] + z_ref[1] + c_ref[0, 0] * xt
    fused = _gelu(_dott(_bf(comb), _bf(fw_ref[...])) + fb_ref[...])
    y_ref[...] = _lnorm(fused + xt, ng_ref[...], nb_ref[...])


def _kfuse(z, xf, cvec, p):
    return pl.pallas_call(
        _kf_body,
        grid=(NT,),
        in_specs=[
            pl.BlockSpec((K, TT, D), lambda t: (0, t, 0)),
            pl.BlockSpec((TT, D), lambda t: (t, 0)),
            pl.BlockSpec(memory_space=pltpu.SMEM),
            pl.BlockSpec((D, D), lambda t: (0, 0)),
            pl.BlockSpec((1, D), lambda t: (0, 0)),
            pl.BlockSpec((1, D), lambda t: (0, 0)),
            pl.BlockSpec((1, D), lambda t: (0, 0)),
        ],
        out_specs=pl.BlockSpec((TT, D), lambda t: (t, 0)),
        out_shape=jax.ShapeDtypeStruct((N, D), jnp.float32),
    )(z, xf, cvec, p['fusion_W'], p['fusion_b'].reshape(1, D),
      p['norm_g'].reshape(1, D), p['norm_b'].reshape(1, D))


def kernel(x, params):
    p = params
    xf = x[0]
    wqh = p['attn_Wq'].reshape(H, DH, D)
    wkh = p['attn_Wk'].reshape(H, DH, D)
    wvh = p['attn_Wv'].reshape(H, DH, D)
    bqh = p['attn_bq'].reshape(H, 1, DH)
    bkh = p['attn_bk'].reshape(H, 1, DH)
    bvh = p['attn_bv'].reshape(H, 1, DH)
    osum = _attn(xf, wqh, wkh, wvh, bqh, bkh, bvh)
    osum2 = osum.reshape(1, D)
    topi, avec, cvec, loss = _gate(xf, osum2, p)
    topi_s = topi.reshape(K)
    itw = _wcast(topi_s, p['it_W'], D)
    l0w = _wcast(topi_s, p['l0_W'], F)
    l1w = _wcast(topi_s, p['l1_W'], FC)
    opw = _wcast(topi_s, p['op_W'], D)
    ig1w = _wcast(topi_s, p['ig1_W'], D // 4)
    ig2w = _wcast(topi_s, p['ig2_W'], D)
    h1, igx = _ka(topi_s, xf, itw, l0w, ig1w, ig2w, p)
    h2pre = _kl1(topi_s, h1, l1w, p)
    z = _kop(topi_s, h2pre, igx, avec, opw, p)
    y = _kfuse(z, xf, cvec, p)
    return y.reshape(1, N, D), loss[0, 0]


# speedup vs baseline: 1.4246x; 1.4246x over previous
import math

import jax
import jax.numpy as jnp
from jax.experimental import pallas as pl
from jax.experimental.pallas import tpu as pltpu

D = 768
F = 3072
E = 8
K = 2
H = 4
DH = D // H
N = 2048
TT = 256
NT = N // TT
CO = 4
FC = F // CO
EPS = 1e-5
HIGH = jax.lax.Precision.HIGHEST


def _dott(a, w, precision=None):
    return jax.lax.dot_general(
        a, w, (((1,), (1,)), ((), ())),
        precision=precision, preferred_element_type=jnp.float32)


def _lnorm(v, g, b):
    m = jnp.mean(v, axis=-1, keepdims=True)
    var = jnp.mean((v - m) ** 2, axis=-1, keepdims=True)
    return (v - m) / jnp.sqrt(var + EPS) * g + b


def _gelu(v):
    return 0.5 * v * (1.0 + jax.lax.erf(v * (1.0 / math.sqrt(2.0))))


def _sp(a):
    hi = a.astype(jnp.bfloat16)
    lo = (a - hi.astype(jnp.float32)).astype(jnp.bfloat16)
    return hi, lo


def _dott3(a, w):
    ahi, alo = _sp(a)
    whi, wlo = _sp(w)
    return _dott(ahi, whi) + _dott(alo, whi) + _dott(ahi, wlo)


def _attn_body(xf_ref, xt_ref, wq_ref, wk_ref, wv_ref, bq_ref, bk_ref,
               bv_ref, osum_ref, khi_ref, klo_ref, vs_ref, cs_ref):
    qt = pl.program_id(1)

    @pl.when(qt == 0)
    def _():
        xf = xf_ref[...]
        kk = _dott3(xf, wk_ref[0]) + bk_ref[0]
        khi, klo = _sp(kk)
        khi_ref[...] = khi
        klo_ref[...] = klo
        vs_ref[...] = _dott3(xf, wv_ref[0]) + bv_ref[0]
        cs_ref[...] = jnp.zeros((1, N), jnp.float32)

    q = _dott3(xt_ref[...], wq_ref[0]) + bq_ref[0]
    qhi, qlo = _sp(q)
    khi = khi_ref[...]
    dims = (((1,), (1,)), ((), ()))
    s = jax.lax.dot_general(qhi, khi, dims, preferred_element_type=jnp.float32)
    s += jax.lax.dot_general(qlo, khi, dims, preferred_element_type=jnp.float32)
    s += jax.lax.dot_general(qhi, klo_ref[...], dims,
                             preferred_element_type=jnp.float32)
    ex = jnp.exp(s * (1.0 / math.sqrt(DH)))
    rs = jnp.sum(ex, axis=-1, keepdims=True)
    prob = ex * (1.0 / rs)
    cs_ref[...] += jnp.sum(prob, axis=0, keepdims=True)

    @pl.when(qt == NT - 1)
    def _():
        osum_ref[0] = jax.lax.dot_general(
            cs_ref[...], vs_ref[...], (((1,), (0,)), ((), ())),
            precision=HIGH, preferred_element_type=jnp.float32)


def _attn(xf, wqh, wkh, wvh, bqh, bkh, bvh):
    return pl.pallas_call(
        _attn_body,
        grid=(H, NT),
        in_specs=[
            pl.BlockSpec((N, D), lambda h, t: (0, 0)),
            pl.BlockSpec((TT, D), lambda h, t: (t, 0)),
            pl.BlockSpec((1, DH, D), lambda h, t: (h, 0, 0)),
            pl.BlockSpec((1, DH, D), lambda h, t: (h, 0, 0)),
            pl.BlockSpec((1, DH, D), lambda h, t: (h, 0, 0)),
            pl.BlockSpec((1, 1, DH), lambda h, t: (h, 0, 0)),
            pl.BlockSpec((1, 1, DH), lambda h, t: (h, 0, 0)),
            pl.BlockSpec((1, 1, DH), lambda h, t: (h, 0, 0)),
        ],
        out_specs=pl.BlockSpec((1, 1, DH), lambda h, t: (h, 0, 0)),
        out_shape=jax.ShapeDtypeStruct((H, 1, DH), jnp.float32),
        scratch_shapes=[pltpu.VMEM((N, DH), jnp.bfloat16),
                        pltpu.VMEM((N, DH), jnp.bfloat16),
                        pltpu.VMEM((N, DH), jnp.float32),
                        pltpu.VMEM((1, N), jnp.float32)],
        compiler_params=pltpu.CompilerParams(
            dimension_semantics=("parallel", "arbitrary")),
    )(xf, xf, wqh, wkh, wvh, bqh, bkh, bvh)


def _gate_body(x_ref, os_ref, wo_ref, bo_ref, g1w_ref, g1b_ref, glg_ref,
               glb_ref, g2w_ref, g2b_ref, g3w_ref, g3b_ref, t1w_ref,
               t1b_ref, t2w_ref, t2b_ref, rw_ref,
               topi_ref, a_ref, c_ref, loss_ref):
    xmean = jnp.mean(x_ref[...], axis=0, keepdims=True)
    pooled = _dott(os_ref[...] * (1.0 / N), wo_ref[...], HIGH) \
        + bo_ref[...] + xmean
    h = _gelu(_dott(pooled, g1w_ref[...], HIGH) + g1b_ref[...])
    h = _lnorm(h, glg_ref[...], glb_ref[...])
    h = _gelu(_dott(h, g2w_ref[...], HIGH) + g2b_ref[...])
    logits = _dott(h, g3w_ref[...], HIGH) + g3b_ref[...]

    t = _gelu(_dott(pooled, t1w_ref[...], HIGH) + t1b_ref[...])
    traw = jnp.sum(t * t2w_ref[...]) + t2b_ref[0]
    temp = jnp.clip(jnp.logaddexp(traw, 0.0), 0.1, 5.0)
    logits = logits / temp

    io8 = jax.lax.broadcasted_iota(jnp.int32, (1, E), 1)
    v1 = jnp.max(logits)
    i1 = jnp.min(jnp.where(logits == v1, io8, E))
    masked = jnp.where(io8 == i1, -jnp.inf, logits)
    v2 = jnp.max(masked)
    i2 = jnp.min(jnp.where(masked == v2, io8, E))

    e2 = jnp.exp(v2 - v1)
    den = 1.0 + e2
    g1v = 1.0 / den
    g2v = e2 / den

    ex = jnp.exp(logits - v1)
    probs = ex / jnp.sum(ex)
    mu = jnp.mean(probs)
    var_loss = jnp.sum((probs - mu) ** 2) / (E - 1) * E
    ent = -jnp.sum(probs * jnp.log(probs + 1e-8))
    load = (var_loss + 0.1 * (math.log(E) - ent)) * 0.01

    rw = rw_ref[...]
    rw1 = jnp.sum(jnp.where(io8 == i1, rw, 0.0))
    rw2 = jnp.sum(jnp.where(io8 == i2, rw, 0.0))

    io2 = jax.lax.broadcasted_iota(jnp.int32, (1, K), 1)
    topi_ref[...] = jnp.where(io2 == 0, i1, i2).astype(jnp.int32)
    a_ref[...] = jnp.where(io2 == 0, g1v * rw1, g2v * rw2)
    c_ref[...] = jnp.reshape(g1v * (1.0 - rw1) + g2v * (1.0 - rw2), (1, 1))
    loss_ref[...] = jnp.reshape(load, (1, 1))


def _gate(xf, osum2, p):
    full = lambda shape: pl.BlockSpec(shape, lambda: tuple(0 for _ in shape))
    args = [xf, osum2, p['attn_Wo'], p['attn_bo'].reshape(1, D),
            p['g1_W'], p['g1_b'].reshape(1, D),
            p['g_ln_g'].reshape(1, D), p['g_ln_b'].reshape(1, D),
            p['g2_W'], p['g2_b'].reshape(1, D // 2),
            p['g3_W'], p['g3_b'].reshape(1, E),
            p['t1_W'], p['t1_b'].reshape(1, D // 4),
            p['t2_W'], p['t2_b'].reshape(1,),
            p['rw'].reshape(1, E)]
    in_specs = [full(a.shape) for a in args]
    in_specs[15] = pl.BlockSpec(memory_space=pltpu.SMEM)
    return pl.pallas_call(
        _gate_body,
        grid=(),
        in_specs=in_specs,
        out_specs=[full((1, K)), full((1, K)), full((1, 1)), full((1, 1))],
        out_shape=[jax.ShapeDtypeStruct((1, K), jnp.int32),
                   jax.ShapeDtypeStruct((1, K), jnp.float32),
                   jax.ShapeDtypeStruct((1, 1), jnp.float32),
                   jax.ShapeDtypeStruct((1, 1), jnp.float32)],
    )(*args)


def _wc_body(s_ref, w_ref, o_ref):
    o_ref[0] = w_ref[0].astype(jnp.bfloat16)


def _wcast(topi, w, rows):
    _, R, C = w.shape
    nc = R // rows
    grid_spec = pltpu.PrefetchScalarGridSpec(
        num_scalar_prefetch=1,
        grid=(K, nc),
        in_specs=[pl.BlockSpec((1, rows, C), lambda k, c, s: (s[k], c, 0))],
        out_specs=pl.BlockSpec((1, rows, C), lambda k, c, s: (k, c, 0)),
    )
    return pl.pallas_call(
        _wc_body,
        grid_spec=grid_spec,
        out_shape=jax.ShapeDtypeStruct((K, R, C), jnp.bfloat16),
        compiler_params=pltpu.CompilerParams(
            dimension_semantics=("arbitrary", "parallel")),
    )(topi, w)


def _bf(v):
    return v.astype(jnp.bfloat16)


def _ka_body(s_ref, x_ref, itw_ref, itb_ref, itg_ref, itb2_ref,
             l0w_ref, l0b_ref, l0g_ref, l0b2_ref,
             ig1w_ref, ig1b_ref, ig2w_ref, ig2b_ref, h1_ref, ig_ref):
    xt = _bf(x_ref[...])
    h0 = _dott(xt, itw_ref[0]) + itb_ref[0]
    h0 = _gelu(_lnorm(h0, itg_ref[0], itb2_ref[0]))
    h1 = _dott(_bf(h0), l0w_ref[0]) + l0b_ref[0]
    h1_ref[0] = _bf(_lnorm(_gelu(h1), l0g_ref[0], l0b2_ref[0]))
    t = _gelu(_dott(xt, ig1w_ref[0]) + ig1b_ref[0])
    ig_ref[0] = _bf(jax.nn.sigmoid(_dott(_bf(t), ig2w_ref[0]) + ig2b_ref[0]))


def _ka(topi, xf, itw, l0w, ig1w, ig2w, p):
    grid_spec = pltpu.PrefetchScalarGridSpec(
        num_scalar_prefetch=1,
        grid=(K, NT),
        in_specs=[
            pl.BlockSpec((TT, D), lambda k, t, s: (t, 0)),
            pl.BlockSpec((1, D, D), lambda k, t, s: (k, 0, 0)),
            pl.BlockSpec((1, 1, D), lambda k, t, s: (s[k], 0, 0)),
            pl.BlockSpec((1, 1, D), lambda k, t, s: (s[k], 0, 0)),
            pl.BlockSpec((1, 1, D), lambda k, t, s: (s[k], 0, 0)),
            pl.BlockSpec((1, F, D), lambda k, t, s: (k, 0, 0)),
            pl.BlockSpec((1, 1, F), lambda k, t, s: (s[k], 0, 0)),
            pl.BlockSpec((1, 1, F), lambda k, t, s: (s[k], 0, 0)),
            pl.BlockSpec((1, 1, F), lambda k, t, s: (s[k], 0, 0)),
            pl.BlockSpec((1, D // 4, D), lambda k, t, s: (k, 0, 0)),
            pl.BlockSpec((1, 1, D // 4), lambda k, t, s: (s[k], 0, 0)),
            pl.BlockSpec((1, D, D // 4), lambda k, t, s: (k, 0, 0)),
            pl.BlockSpec((1, 1, D), lambda k, t, s: (s[k], 0, 0)),
        ],
        out_specs=[pl.BlockSpec((1, TT, F), lambda k, t, s: (k, t, 0)),
                   pl.BlockSpec((1, TT, D), lambda k, t, s: (k, t, 0))],
    )
    return pl.pallas_call(
        _ka_body,
        grid_spec=grid_spec,
        out_shape=[jax.ShapeDtypeStruct((K, N, F), jnp.bfloat16),
                   jax.ShapeDtypeStruct((K, N, D), jnp.bfloat16)],
        compiler_params=pltpu.CompilerParams(
            dimension_semantics=("arbitrary", "parallel")),
    )(topi, xf, itw, p['it_b'].reshape(E, 1, D),
      p['it_ln_g'].reshape(E, 1, D), p['it_ln_b'].reshape(E, 1, D),
      l0w, p['l0_b'].reshape(E, 1, F),
      p['l0_ln_g'].reshape(E, 1, F), p['l0_ln_b'].reshape(E, 1, F),
      ig1w, p['ig1_b'].reshape(E, 1, D // 4),
      ig2w, p['ig2_b'].reshape(E, 1, D))


def _kl1_body(s_ref, h1_ref, w_ref, b_ref, out_ref):
    out_ref[0] = _bf(_dott(h1_ref[0], w_ref[0]) + b_ref[0])


def _kl1(topi, h1, l1w, p):
    grid_spec = pltpu.PrefetchScalarGridSpec(
        num_scalar_prefetch=1,
        grid=(K, CO, NT),
        in_specs=[
            pl.BlockSpec((1, TT, F), lambda k, c, t, s: (k, t, 0)),
            pl.BlockSpec((1, FC, F), lambda k, c, t, s: (k, c, 0)),
            pl.BlockSpec((1, 1, FC), lambda k, c, t, s: (s[k], 0, c)),
        ],
        out_specs=pl.BlockSpec((1, TT, FC), lambda k, c, t, s: (k, t, c)),
    )
    return pl.pallas_call(
        _kl1_body,
        grid_spec=grid_spec,
        out_shape=jax.ShapeDtypeStruct((K, N, F), jnp.bfloat16),
        compiler_params=pltpu.CompilerParams(
            dimension_semantics=("arbitrary", "arbitrary", "parallel")),
    )(topi, h1, l1w, p['l1_b'].reshape(E, 1, F))


def _kop_body(s_ref, h2p_ref, l1g_ref, l1b2_ref, opw_ref, opb_ref,
              ig_ref, es_ref, eb_ref, a_ref, z_ref):
    h2 = _lnorm(_gelu(h2p_ref[0].astype(jnp.float32)), l1g_ref[0], l1b2_ref[0])
    o = _dott(_bf(h2), opw_ref[0]) + opb_ref[0]
    o = o * ig_ref[0].astype(jnp.float32) * es_ref[0] + eb_ref[0]
    k = pl.program_id(0)
    z_ref[0] = a_ref[0, k] * o


def _kop(topi, h2pre, igx, avec, opw, p):
    grid_spec = pltpu.PrefetchScalarGridSpec(
        num_scalar_prefetch=1,
        grid=(K, NT),
        in_specs=[
            pl.BlockSpec((1, TT, F), lambda k, t, s: (k, t, 0)),
            pl.BlockSpec((1, 1, F), lambda k, t, s: (s[k], 0, 0)),
            pl.BlockSpec((1, 1, F), lambda k, t, s: (s[k], 0, 0)),
            pl.BlockSpec((1, D, F), lambda k, t, s: (k, 0, 0)),
            pl.BlockSpec((1, 1, D), lambda k, t, s: (s[k], 0, 0)),
            pl.BlockSpec((1, TT, D), lambda k, t, s: (k, t, 0)),
            pl.BlockSpec((1, 1, D), lambda k, t, s: (s[k], 0, 0)),
            pl.BlockSpec((1, 1, D), lambda k, t, s: (s[k], 0, 0)),
            pl.BlockSpec(memory_space=pltpu.SMEM),
        ],
        out_specs=pl.BlockSpec((1, TT, D), lambda k, t, s: (k, t, 0)),
    )
    return pl.pallas_call(
        _kop_body,
        grid_spec=grid_spec,
        out_shape=jax.ShapeDtypeStruct((K, N, D), jnp.float32),
        compiler_params=pltpu.CompilerParams(
            dimension_semantics=("arbitrary", "parallel")),
    )(topi, h2pre, p['l1_ln_g'].reshape(E, 1, F),
      p['l1_ln_b'].reshape(E, 1, F),
      opw, p['op_b'].reshape(E, 1, D),
      igx, p['es'].reshape(E, 1, D), p['eb'].reshape(E, 1, D), avec)


def _kf_body(z_ref, x_ref, c_ref, fw_ref, fb_ref, ng_ref, nb_ref, y_ref):
    xt = x_ref[...]
    comb = z_ref[0] + z_ref[1] + c_ref[0, 0] * xt
    fused = _gelu(_dott(_bf(comb), _bf(fw_ref[...])) + fb_ref[...])
    y_ref[...] = _lnorm(fused + xt, ng_ref[...], nb_ref[...])


def _kfuse(z, xf, cvec, p):
    return pl.pallas_call(
        _kf_body,
        grid=(NT,),
        in_specs=[
            pl.BlockSpec((K, TT, D), lambda t: (0, t, 0)),
            pl.BlockSpec((TT, D), lambda t: (t, 0)),
            pl.BlockSpec(memory_space=pltpu.SMEM),
            pl.BlockSpec((D, D), lambda t: (0, 0)),
            pl.BlockSpec((1, D), lambda t: (0, 0)),
            pl.BlockSpec((1, D), lambda t: (0, 0)),
            pl.BlockSpec((1, D), lambda t: (0, 0)),
        ],
        out_specs=pl.BlockSpec((TT, D), lambda t: (t, 0)),
        out_shape=jax.ShapeDtypeStruct((N, D), jnp.float32),
        compiler_params=pltpu.CompilerParams(
            dimension_semantics=("parallel",)),
    )(z, xf, cvec, p['fusion_W'], p['fusion_b'].reshape(1, D),
      p['norm_g'].reshape(1, D), p['norm_b'].reshape(1, D))


def kernel(x, params):
    p = params
    xf = x[0]
    wqh = p['attn_Wq'].reshape(H, DH, D)
    wkh = p['attn_Wk'].reshape(H, DH, D)
    wvh = p['attn_Wv'].reshape(H, DH, D)
    bqh = p['attn_bq'].reshape(H, 1, DH)
    bkh = p['attn_bk'].reshape(H, 1, DH)
    bvh = p['attn_bv'].reshape(H, 1, DH)
    osum = _attn(xf, wqh, wkh, wvh, bqh, bkh, bvh)
    osum2 = osum.reshape(1, D)
    topi, avec, cvec, loss = _gate(xf, osum2, p)
    topi_s = topi.reshape(K)
    itw = _wcast(topi_s, p['it_W'], D)
    l0w = _wcast(topi_s, p['l0_W'], F)
    l1w = _wcast(topi_s, p['l1_W'], FC)
    opw = _wcast(topi_s, p['op_W'], D)
    ig1w = _wcast(topi_s, p['ig1_W'], D // 4)
    ig2w = _wcast(topi_s, p['ig2_W'], D)
    h1, igx = _ka(topi_s, xf, itw, l0w, ig1w, ig2w, p)
    h2pre = _kl1(topi_s, h1, l1w, p)
    z = _kop(topi_s, h2pre, igx, avec, opw, p)
    y = _kfuse(z, xf, cvec, p)
    return y.reshape(1, N, D), loss[0, 0]


# speedup vs baseline: 1.6088x; 1.1293x over previous
import math

import jax
import jax.numpy as jnp
from jax.experimental import pallas as pl
from jax.experimental.pallas import tpu as pltpu

D = 768
F = 3072
E = 8
K = 2
H = 4
DH = D // H
N = 2048
TT = 256
TQ = 512
NT = N // TT
CO = 4
FC = F // CO
EPS = 1e-5
HIGH = jax.lax.Precision.HIGHEST


def _dott(a, w, precision=None):
    return jax.lax.dot_general(
        a, w, (((1,), (1,)), ((), ())),
        precision=precision, preferred_element_type=jnp.float32)


def _lnorm(v, g, b):
    m = jnp.mean(v, axis=-1, keepdims=True)
    var = jnp.mean((v - m) ** 2, axis=-1, keepdims=True)
    return (v - m) / jnp.sqrt(var + EPS) * g + b


def _gelu(v):
    return 0.5 * v * (1.0 + jax.lax.erf(v * (1.0 / math.sqrt(2.0))))


def _sp(a):
    hi = a.astype(jnp.bfloat16)
    lo = (a - hi.astype(jnp.float32)).astype(jnp.bfloat16)
    return hi, lo


def _dott3(a, w):
    ahi, alo = _sp(a)
    whi, wlo = _sp(w)
    return _dott(ahi, whi) + _dott(alo, whi) + _dott(ahi, wlo)


def _attn_body(xf_ref, xt_ref, wq_ref, wk_ref, wv_ref, bq_ref, bk_ref,
               bv_ref, osum_ref, xmean_ref, khi_ref, klo_ref, vs_ref,
               cs_ref):
    h = pl.program_id(0)
    qt = pl.program_id(1)

    @pl.when((h == 0) & (qt == 0))
    def _():
        xmean_ref[...] = jnp.sum(xf_ref[...], axis=0, keepdims=True) * (1.0 / N)

    @pl.when(qt == 0)
    def _():
        xf = xf_ref[...]
        kk = _dott3(xf, wk_ref[0]) + bk_ref[0]
        khi, klo = _sp(kk)
        khi_ref[...] = khi
        klo_ref[...] = klo
        vs_ref[...] = _dott3(xf, wv_ref[0]) + bv_ref[0]
        cs_ref[...] = jnp.zeros((1, N), jnp.float32)

    q = _dott3(xt_ref[...], wq_ref[0]) + bq_ref[0]
    qhi, qlo = _sp(q)
    khi = khi_ref[...]
    dims = (((1,), (1,)), ((), ()))
    s = jax.lax.dot_general(qhi, khi, dims, preferred_element_type=jnp.float32)
    s += jax.lax.dot_general(qlo, khi, dims, preferred_element_type=jnp.float32)
    s += jax.lax.dot_general(qhi, klo_ref[...], dims,
                             preferred_element_type=jnp.float32)
    ex = jnp.exp(s * (1.0 / math.sqrt(DH)))
    rs = jnp.sum(ex, axis=-1, keepdims=True)
    prob = ex * (1.0 / rs)
    cs_ref[...] += jnp.sum(prob, axis=0, keepdims=True)

    @pl.when(qt == N // TQ - 1)
    def _():
        osum_ref[0] = jax.lax.dot_general(
            cs_ref[...], vs_ref[...], (((1,), (0,)), ((), ())),
            precision=HIGH, preferred_element_type=jnp.float32)


def _attn(xf, wqh, wkh, wvh, bqh, bkh, bvh):
    return pl.pallas_call(
        _attn_body,
        grid=(H, N // TQ),
        in_specs=[
            pl.BlockSpec((N, D), lambda h, t: (0, 0)),
            pl.BlockSpec((TQ, D), lambda h, t: (t, 0)),
            pl.BlockSpec((1, DH, D), lambda h, t: (h, 0, 0)),
            pl.BlockSpec((1, DH, D), lambda h, t: (h, 0, 0)),
            pl.BlockSpec((1, DH, D), lambda h, t: (h, 0, 0)),
            pl.BlockSpec((1, 1, DH), lambda h, t: (h, 0, 0)),
            pl.BlockSpec((1, 1, DH), lambda h, t: (h, 0, 0)),
            pl.BlockSpec((1, 1, DH), lambda h, t: (h, 0, 0)),
        ],
        out_specs=[pl.BlockSpec((1, 1, DH), lambda h, t: (h, 0, 0)),
                   pl.BlockSpec((1, D), lambda h, t: (0, 0))],
        out_shape=[jax.ShapeDtypeStruct((H, 1, DH), jnp.float32),
                   jax.ShapeDtypeStruct((1, D), jnp.float32)],
        scratch_shapes=[pltpu.VMEM((N, DH), jnp.bfloat16),
                        pltpu.VMEM((N, DH), jnp.bfloat16),
                        pltpu.VMEM((N, DH), jnp.float32),
                        pltpu.VMEM((1, N), jnp.float32)],
        compiler_params=pltpu.CompilerParams(
            dimension_semantics=("parallel", "arbitrary")),
    )(xf, xf, wqh, wkh, wvh, bqh, bkh, bvh)


def _gate_body(xm_ref, os_ref, wo_ref, bo_ref, g1w_ref, g1b_ref, glg_ref,
               glb_ref, g2w_ref, g2b_ref, g3w_ref, g3b_ref, t1w_ref,
               t1b_ref, t2w_ref, t2b_ref, rw_ref,
               topi_ref, a_ref, c_ref, loss_ref):
    pooled = _dott(os_ref[...] * (1.0 / N), wo_ref[...], HIGH) \
        + bo_ref[...] + xm_ref[...]
    h = _gelu(_dott(pooled, g1w_ref[...], HIGH) + g1b_ref[...])
    h = _lnorm(h, glg_ref[...], glb_ref[...])
    h = _gelu(_dott(h, g2w_ref[...], HIGH) + g2b_ref[...])
    logits = _dott(h, g3w_ref[...], HIGH) + g3b_ref[...]

    t = _gelu(_dott(pooled, t1w_ref[...], HIGH) + t1b_ref[...])
    traw = jnp.sum(t * t2w_ref[...]) + t2b_ref[0]
    temp = jnp.clip(jnp.logaddexp(traw, 0.0), 0.1, 5.0)
    logits = logits / temp

    io8 = jax.lax.broadcasted_iota(jnp.int32, (1, E), 1)
    v1 = jnp.max(logits)
    i1 = jnp.min(jnp.where(logits == v1, io8, E))
    masked = jnp.where(io8 == i1, -jnp.inf, logits)
    v2 = jnp.max(masked)
    i2 = jnp.min(jnp.where(masked == v2, io8, E))

    e2 = jnp.exp(v2 - v1)
    den = 1.0 + e2
    g1v = 1.0 / den
    g2v = e2 / den

    ex = jnp.exp(logits - v1)
    probs = ex / jnp.sum(ex)
    mu = jnp.mean(probs)
    var_loss = jnp.sum((probs - mu) ** 2) / (E - 1) * E
    ent = -jnp.sum(probs * jnp.log(probs + 1e-8))
    load = (var_loss + 0.1 * (math.log(E) - ent)) * 0.01

    rw = rw_ref[...]
    rw1 = jnp.sum(jnp.where(io8 == i1, rw, 0.0))
    rw2 = jnp.sum(jnp.where(io8 == i2, rw, 0.0))

    io2 = jax.lax.broadcasted_iota(jnp.int32, (1, K), 1)
    topi_ref[...] = jnp.where(io2 == 0, i1, i2).astype(jnp.int32)
    a_ref[...] = jnp.where(io2 == 0, g1v * rw1, g2v * rw2)
    c_ref[...] = jnp.reshape(g1v * (1.0 - rw1) + g2v * (1.0 - rw2), (1, 1))
    loss_ref[...] = jnp.reshape(load, (1, 1))


def _gate(xmean, osum2, p):
    full = lambda shape: pl.BlockSpec(shape, lambda: tuple(0 for _ in shape))
    args = [xmean, osum2, p['attn_Wo'], p['attn_bo'].reshape(1, D),
            p['g1_W'], p['g1_b'].reshape(1, D),
            p['g_ln_g'].reshape(1, D), p['g_ln_b'].reshape(1, D),
            p['g2_W'], p['g2_b'].reshape(1, D // 2),
            p['g3_W'], p['g3_b'].reshape(1, E),
            p['t1_W'], p['t1_b'].reshape(1, D // 4),
            p['t2_W'], p['t2_b'].reshape(1,),
            p['rw'].reshape(1, E)]
    in_specs = [full(a.shape) for a in args]
    in_specs[15] = pl.BlockSpec(memory_space=pltpu.SMEM)
    return pl.pallas_call(
        _gate_body,
        grid=(),
        in_specs=in_specs,
        out_specs=[full((1, K)), full((1, K)), full((1, 1)), full((1, 1))],
        out_shape=[jax.ShapeDtypeStruct((1, K), jnp.int32),
                   jax.ShapeDtypeStruct((1, K), jnp.float32),
                   jax.ShapeDtypeStruct((1, 1), jnp.float32),
                   jax.ShapeDtypeStruct((1, 1), jnp.float32)],
    )(*args)


def _bf(v):
    return v.astype(jnp.bfloat16)


def _ka_body(s_ref, x_ref, itw_ref, itb_ref, itg_ref, itb2_ref,
             l0w_ref, l0b_ref, l0g_ref, l0b2_ref,
             ig1w_ref, ig1b_ref, ig2w_ref, ig2b_ref, h1_ref, ig_ref,
             itws, l0ws, ig1ws, ig2ws):
    t = pl.program_id(1)

    @pl.when(t == 0)
    def _():
        itws[...] = _bf(itw_ref[0])
        l0ws[...] = _bf(l0w_ref[0])
        ig1ws[...] = _bf(ig1w_ref[0])
        ig2ws[...] = _bf(ig2w_ref[0])

    xt = _bf(x_ref[...])
    h0 = _dott(xt, itws[...]) + itb_ref[0]
    h0 = _gelu(_lnorm(h0, itg_ref[0], itb2_ref[0]))
    h1 = _dott(_bf(h0), l0ws[...]) + l0b_ref[0]
    h1_ref[0] = _bf(_lnorm(_gelu(h1), l0g_ref[0], l0b2_ref[0]))
    tt = _gelu(_dott(xt, ig1ws[...]) + ig1b_ref[0])
    ig_ref[0] = _bf(jax.nn.sigmoid(_dott(_bf(tt), ig2ws[...]) + ig2b_ref[0]))


def _ka(topi, xf, p):
    grid_spec = pltpu.PrefetchScalarGridSpec(
        num_scalar_prefetch=1,
        grid=(K, NT),
        in_specs=[
            pl.BlockSpec((TT, D), lambda k, t, s: (t, 0)),
            pl.BlockSpec((1, D, D), lambda k, t, s: (s[k], 0, 0)),
            pl.BlockSpec((1, 1, D), lambda k, t, s: (s[k], 0, 0)),
            pl.BlockSpec((1, 1, D), lambda k, t, s: (s[k], 0, 0)),
            pl.BlockSpec((1, 1, D), lambda k, t, s: (s[k], 0, 0)),
            pl.BlockSpec((1, F, D), lambda k, t, s: (s[k], 0, 0)),
            pl.BlockSpec((1, 1, F), lambda k, t, s: (s[k], 0, 0)),
            pl.BlockSpec((1, 1, F), lambda k, t, s: (s[k], 0, 0)),
            pl.BlockSpec((1, 1, F), lambda k, t, s: (s[k], 0, 0)),
            pl.BlockSpec((1, D // 4, D), lambda k, t, s: (s[k], 0, 0)),
            pl.BlockSpec((1, 1, D // 4), lambda k, t, s: (s[k], 0, 0)),
            pl.BlockSpec((1, D, D // 4), lambda k, t, s: (s[k], 0, 0)),
            pl.BlockSpec((1, 1, D), lambda k, t, s: (s[k], 0, 0)),
        ],
        out_specs=[pl.BlockSpec((1, TT, F), lambda k, t, s: (k, t, 0)),
                   pl.BlockSpec((1, TT, D), lambda k, t, s: (k, t, 0))],
        scratch_shapes=[pltpu.VMEM((D, D), jnp.bfloat16),
                        pltpu.VMEM((F, D), jnp.bfloat16),
                        pltpu.VMEM((D // 4, D), jnp.bfloat16),
                        pltpu.VMEM((D, D // 4), jnp.bfloat16)],
    )
    return pl.pallas_call(
        _ka_body,
        grid_spec=grid_spec,
        out_shape=[jax.ShapeDtypeStruct((K, N, F), jnp.bfloat16),
                   jax.ShapeDtypeStruct((K, N, D), jnp.bfloat16)],
        compiler_params=pltpu.CompilerParams(
            dimension_semantics=("arbitrary", "arbitrary")),
    )(topi, xf, p['it_W'], p['it_b'].reshape(E, 1, D),
      p['it_ln_g'].reshape(E, 1, D), p['it_ln_b'].reshape(E, 1, D),
      p['l0_W'], p['l0_b'].reshape(E, 1, F),
      p['l0_ln_g'].reshape(E, 1, F), p['l0_ln_b'].reshape(E, 1, F),
      p['ig1_W'], p['ig1_b'].reshape(E, 1, D // 4),
      p['ig2_W'], p['ig2_b'].reshape(E, 1, D))


def _kl1_body(s_ref, h1_ref, w_ref, b_ref, out_ref, ws):
    t = pl.program_id(2)

    @pl.when(t == 0)
    def _():
        ws[...] = _bf(w_ref[0])

    out_ref[0] = _bf(_dott(h1_ref[0], ws[...]) + b_ref[0])


def _kl1(topi, h1, p):
    grid_spec = pltpu.PrefetchScalarGridSpec(
        num_scalar_prefetch=1,
        grid=(K, CO, NT),
        in_specs=[
            pl.BlockSpec((1, TT, F), lambda k, c, t, s: (k, t, 0)),
            pl.BlockSpec((1, FC, F), lambda k, c, t, s: (s[k], c, 0)),
            pl.BlockSpec((1, 1, FC), lambda k, c, t, s: (s[k], 0, c)),
        ],
        out_specs=pl.BlockSpec((1, TT, FC), lambda k, c, t, s: (k, t, c)),
        scratch_shapes=[pltpu.VMEM((FC, F), jnp.bfloat16)],
    )
    return pl.pallas_call(
        _kl1_body,
        grid_spec=grid_spec,
        out_shape=jax.ShapeDtypeStruct((K, N, F), jnp.bfloat16),
        compiler_params=pltpu.CompilerParams(
            dimension_semantics=("arbitrary", "arbitrary", "arbitrary")),
    )(topi, h1, p['l1_W'], p['l1_b'].reshape(E, 1, F))


def _kop_body(s_ref, h2p_ref, l1g_ref, l1b2_ref, opw_ref, opb_ref,
              ig_ref, es_ref, eb_ref, a_ref, z_ref, ws):
    t = pl.program_id(1)

    @pl.when(t == 0)
    def _():
        ws[...] = _bf(opw_ref[0])

    h2 = _lnorm(_gelu(h2p_ref[0].astype(jnp.float32)), l1g_ref[0], l1b2_ref[0])
    o = _dott(_bf(h2), ws[...]) + opb_ref[0]
    o = o * ig_ref[0].astype(jnp.float32) * es_ref[0] + eb_ref[0]
    k = pl.program_id(0)
    z_ref[0] = _bf(a_ref[0, k] * o)


def _kop(topi, h2pre, igx, avec, p):
    grid_spec = pltpu.PrefetchScalarGridSpec(
        num_scalar_prefetch=1,
        grid=(K, NT),
        in_specs=[
            pl.BlockSpec((1, TT, F), lambda k, t, s: (k, t, 0)),
            pl.BlockSpec((1, 1, F), lambda k, t, s: (s[k], 0, 0)),
            pl.BlockSpec((1, 1, F), lambda k, t, s: (s[k], 0, 0)),
            pl.BlockSpec((1, D, F), lambda k, t, s: (s[k], 0, 0)),
            pl.BlockSpec((1, 1, D), lambda k, t, s: (s[k], 0, 0)),
            pl.BlockSpec((1, TT, D), lambda k, t, s: (k, t, 0)),
            pl.BlockSpec((1, 1, D), lambda k, t, s: (s[k], 0, 0)),
            pl.BlockSpec((1, 1, D), lambda k, t, s: (s[k], 0, 0)),
            pl.BlockSpec(memory_space=pltpu.SMEM),
        ],
        out_specs=pl.BlockSpec((1, TT, D), lambda k, t, s: (k, t, 0)),
        scratch_shapes=[pltpu.VMEM((D, F), jnp.bfloat16)],
    )
    return pl.pallas_call(
        _kop_body,
        grid_spec=grid_spec,
        out_shape=jax.ShapeDtypeStruct((K, N, D), jnp.bfloat16),
        compiler_params=pltpu.CompilerParams(
            dimension_semantics=("arbitrary", "arbitrary")),
    )(topi, h2pre, p['l1_ln_g'].reshape(E, 1, F),
      p['l1_ln_b'].reshape(E, 1, F),
      p['op_W'], p['op_b'].reshape(E, 1, D),
      igx, p['es'].reshape(E, 1, D), p['eb'].reshape(E, 1, D), avec)


def _kf_body(z_ref, x_ref, c_ref, fw_ref, fb_ref, ng_ref, nb_ref, y_ref):
    xt = x_ref[...]
    comb = (z_ref[0].astype(jnp.float32) + z_ref[1].astype(jnp.float32)
            + c_ref[0, 0] * xt)
    fused = _gelu(_dott(_bf(comb), _bf(fw_ref[...])) + fb_ref[...])
    y_ref[...] = _lnorm(fused + xt, ng_ref[...], nb_ref[...])


def _kfuse(z, xf, cvec, p):
    return pl.pallas_call(
        _kf_body,
        grid=(NT,),
        in_specs=[
            pl.BlockSpec((K, TT, D), lambda t: (0, t, 0)),
            pl.BlockSpec((TT, D), lambda t: (t, 0)),
            pl.BlockSpec(memory_space=pltpu.SMEM),
            pl.BlockSpec((D, D), lambda t: (0, 0)),
            pl.BlockSpec((1, D), lambda t: (0, 0)),
            pl.BlockSpec((1, D), lambda t: (0, 0)),
            pl.BlockSpec((1, D), lambda t: (0, 0)),
        ],
        out_specs=pl.BlockSpec((TT, D), lambda t: (t, 0)),
        out_shape=jax.ShapeDtypeStruct((N, D), jnp.float32),
        compiler_params=pltpu.CompilerParams(
            dimension_semantics=("parallel",)),
    )(z, xf, cvec, p['fusion_W'], p['fusion_b'].reshape(1, D),
      p['norm_g'].reshape(1, D), p['norm_b'].reshape(1, D))


def kernel(x, params):
    p = params
    xf = x[0]
    wqh = p['attn_Wq'].reshape(H, DH, D)
    wkh = p['attn_Wk'].reshape(H, DH, D)
    wvh = p['attn_Wv'].reshape(H, DH, D)
    bqh = p['attn_bq'].reshape(H, 1, DH)
    bkh = p['attn_bk'].reshape(H, 1, DH)
    bvh = p['attn_bv'].reshape(H, 1, DH)
    osum, xmean = _attn(xf, wqh, wkh, wvh, bqh, bkh, bvh)
    osum2 = osum.reshape(1, D)
    topi, avec, cvec, loss = _gate(xmean, osum2, p)
    topi_s = topi.reshape(K)
    h1, igx = _ka(topi_s, xf, p)
    h2pre = _kl1(topi_s, h1, p)
    z = _kop(topi_s, h2pre, igx, avec, p)
    y = _kfuse(z, xf, cvec, p)
    return y.reshape(1, N, D), loss[0, 0]
